# Initial kernel scaffold; baseline (speedup 1.0000x reference)
import jax
import jax.numpy as jnp
from jax.experimental import pallas as pl


def kernel(x, edge_index, bot_up_edge_index, root_feat, text_feat, batch, W_td1, b_td1, W_td2, b_td2, W_bu1, b_bu1, W_bu2, b_bu2, W_h1, b_h1, W_h2, b_h2):
    raise NotImplementedError("write your pallas kernel here")


# speedup vs baseline: 1.6554x; 1.6554x over previous
import functools

import jax
import jax.numpy as jnp
from jax import lax
from jax.experimental import pallas as pl
from jax.experimental.pallas import tpu as pltpu

N = 10000
E = 320000
G = 128
XD = 16
TD = 112
IND = 128
HID = 256
NCLS = 4
CH = 1000
NB = N // CH
DEGP = 8


def _prep_body(x_ref, tf_ref, rt_ref, b3_ref, dtd_ref, dbu_ref,
               utd_ref, ubu_ref, rf_ref, ditd_ref, dibu_ref):
    deg_td = jnp.sum(dtd_ref[0].astype(jnp.float32), axis=0) + 1.0
    deg_bu = jnp.sum(dbu_ref[0].astype(jnp.float32), axis=0) + 1.0
    di_td = lax.rsqrt(deg_td)[:, None]
    di_bu = lax.rsqrt(deg_bu)[:, None]
    batch = b3_ref[0, 0, :]
    gid = lax.broadcasted_iota(jnp.int32, (CH, G), 1)
    onehot = (batch[:, None] == gid).astype(jnp.float32)
    tf = jnp.dot(onehot, tf_ref[...], preferred_element_type=jnp.float32)
    rg = jnp.dot(onehot, rt_ref[...], preferred_element_type=jnp.float32)
    xf = jnp.concatenate([x_ref[...], tf], axis=1)
    rf_ref[...] = jnp.concatenate([rg, tf], axis=1)
    utd = xf * di_td
    ubu = xf * di_bu
    utd_ref[0] = utd[:, :64]
    utd_ref[1] = utd[:, 64:]
    ubu_ref[0] = ubu[:, :64]
    ubu_ref[1] = ubu[:, 64:]
    ditd_ref[...] = jnp.broadcast_to(di_td, (CH, 8))
    dibu_ref[...] = jnp.broadcast_to(di_bu, (CH, 8))


def _prep(x, text_feat, root_feat, batch3, degp_td, degp_bu):
    f32 = jnp.float32
    return pl.pallas_call(
        _prep_body,
        grid=(NB,),
        in_specs=[
            pl.BlockSpec((CH, XD), lambda i: (i, 0)),
            pl.BlockSpec((G, TD), lambda i: (0, 0)),
            pl.BlockSpec((G, XD), lambda i: (0, 0)),
            pl.BlockSpec((1, 1, CH), lambda i: (i, 0, 0)),
            pl.BlockSpec((1, DEGP, CH), lambda i: (i, 0, 0)),
            pl.BlockSpec((1, DEGP, CH), lambda i: (i, 0, 0)),
        ],
        out_specs=[
            pl.BlockSpec((2, CH, 64), lambda i: (0, i, 0)),
            pl.BlockSpec((2, CH, 64), lambda i: (0, i, 0)),
            pl.BlockSpec((CH, IND), lambda i: (i, 0)),
            pl.BlockSpec((CH, 8), lambda i: (i, 0)),
            pl.BlockSpec((CH, 8), lambda i: (i, 0)),
        ],
        out_shape=[
            jax.ShapeDtypeStruct((2, N, 64), f32),
            jax.ShapeDtypeStruct((2, N, 64), f32),
            jax.ShapeDtypeStruct((N, IND), f32),
            jax.ShapeDtypeStruct((N, 8), f32),
            jax.ShapeDtypeStruct((N, 8), f32),
        ],
    )(x, text_feat, root_feat, batch3, degp_td, degp_bu)


def _mid_body(stdl_ref, stdh_ref, utdl_ref, utdh_ref,
              sbul_ref, sbuh_ref, ubul_ref, ubuh_ref,
              rf_ref, ditd_ref, dibu_ref,
              wtd1_ref, btd1_ref, wtd2a_ref, wtd2b_ref,
              wbu1_ref, bbu1_ref, wbu2a_ref, wbu2b_ref,
              otd_ref, obu_ref):
    rf = rf_ref[...]

    def one_dir(slo, shi, ulo, uhi, di_ref, w1, b1, w2a, w2b, out_ref):
        s = jnp.concatenate([slo[0], shi[0]], axis=1)
        u = jnp.concatenate([ulo[0], uhi[0]], axis=1)
        di = di_ref[:, 0:1]
        y = di * (s + u)
        xh = jax.nn.relu(
            jnp.dot(y, w1[...], preferred_element_type=jnp.float32) + b1[0])
        u2 = di * (jnp.dot(xh, w2a[...], preferred_element_type=jnp.float32)
                   + jnp.dot(rf, w2b[...], preferred_element_type=jnp.float32))
        out_ref[0] = u2[:, :128]
        out_ref[1] = u2[:, 128:]

    one_dir(stdl_ref, stdh_ref, utdl_ref, utdh_ref, ditd_ref,
            wtd1_ref, btd1_ref, wtd2a_ref, wtd2b_ref, otd_ref)
    one_dir(sbul_ref, sbuh_ref, ubul_ref, ubuh_ref, dibu_ref,
            wbu1_ref, bbu1_ref, wbu2a_ref, wbu2b_ref, obu_ref)


def _mid(s_td1, u_td1, s_bu1, u_bu1, rf, ditd, dibu,
         W_td1, b_td1, W_td2a, W_td2b, W_bu1, b_bu1, W_bu2a, W_bu2b):
    f32 = jnp.float32
    half = lambda j: pl.BlockSpec((1, CH, 64), lambda i, j=j: (j, i, 0))
    full = lambda a, b: pl.BlockSpec((a, b), lambda i: (0, 0))
    return pl.pallas_call(
        _mid_body,
        grid=(NB,),
        in_specs=[
            half(0), half(1), half(0), half(1),
            half(0), half(1), half(0), half(1),
            pl.BlockSpec((CH, IND), lambda i: (i, 0)),
            pl.BlockSpec((CH, 8), lambda i: (i, 0)),
            pl.BlockSpec((CH, 8), lambda i: (i, 0)),
            full(IND, HID), pl.BlockSpec((1, HID), lambda i: (0, 0)),
            full(HID, HID), full(IND, HID),
            full(IND, HID), pl.BlockSpec((1, HID), lambda i: (0, 0)),
            full(HID, HID), full(IND, HID),
        ],
        out_specs=[
            pl.BlockSpec((2, CH, 128), lambda i: (0, i, 0)),
            pl.BlockSpec((2, CH, 128), lambda i: (0, i, 0)),
        ],
        out_shape=[
            jax.ShapeDtypeStruct((2, N, 128), f32),
            jax.ShapeDtypeStruct((2, N, 128), f32),
        ],
    )(s_td1, s_td1, u_td1, u_td1, s_bu1, s_bu1, u_bu1, u_bu1,
      rf, ditd, dibu, W_td1, b_td1, W_td2a, W_td2b,
      W_bu1, b_bu1, W_bu2a, W_bu2b)


def _final_body(stdl_ref, stdh_ref, utdl_ref, utdh_ref,
                sbul_ref, sbuh_ref, ubul_ref, ubuh_ref,
                ditd_ref, dibu_ref, btd2_ref, bbu2_ref, b3_ref,
                wh1_ref, bh1_ref, wh2_ref, bh2_ref,
                out_ref, acc_td, acc_bu, cnt):
    i = pl.program_id(0)

    @pl.when(i == 0)
    def _():
        acc_td[...] = jnp.zeros_like(acc_td)
        acc_bu[...] = jnp.zeros_like(acc_bu)
        cnt[...] = jnp.zeros_like(cnt)

    batch = b3_ref[0, 0, :]
    gid = lax.broadcasted_iota(jnp.int32, (G, CH), 0)
    mask = (batch[None, :] == gid).astype(jnp.float32)

    def one_dir(slo, shi, ulo, uhi, di_ref, b2):
        s = jnp.concatenate([slo[0], shi[0]], axis=1)
        u = jnp.concatenate([ulo[0], uhi[0]], axis=1)
        return jax.nn.relu(di_ref[:, 0:1] * (s + u) + b2[0])

    xtd2 = one_dir(stdl_ref, stdh_ref, utdl_ref, utdh_ref, ditd_ref, btd2_ref)
    xbu2 = one_dir(sbul_ref, sbuh_ref, ubul_ref, ubuh_ref, dibu_ref, bbu2_ref)
    acc_td[...] += jnp.dot(mask, xtd2, preferred_element_type=jnp.float32)
    acc_bu[...] += jnp.dot(mask, xbu2, preferred_element_type=jnp.float32)
    cnt[...] += jnp.dot(mask, jnp.ones((CH, 128), jnp.float32),
                        preferred_element_type=jnp.float32)

    @pl.when(i == NB - 1)
    def _():
        c = jnp.maximum(cnt[:, 0:1], 1.0)
        pooled = jnp.concatenate([acc_td[...] / c, acc_bu[...] / c], axis=1)
        h = jax.nn.relu(
            jnp.dot(pooled, wh1_ref[...], preferred_element_type=jnp.float32)
            + bh1_ref[0])
        out_ref[...] = (jnp.dot(h, wh2_ref[...],
                                preferred_element_type=jnp.float32)
                        + bh2_ref[0])


def _final(s_td2, u_td2, s_bu2, u_bu2, ditd, dibu, b_td2, b_bu2, batch3,
           W_h1, b_h1, W_h2p, b_h2p):
    f32 = jnp.float32
    half = lambda j: pl.BlockSpec((1, CH, 128), lambda i, j=j: (j, i, 0))
    return pl.pallas_call(
        _final_body,
        grid=(NB,),
        in_specs=[
            half(0), half(1), half(0), half(1),
            half(0), half(1), half(0), half(1),
            pl.BlockSpec((CH, 8), lambda i: (i, 0)),
            pl.BlockSpec((CH, 8), lambda i: (i, 0)),
            pl.BlockSpec((1, HID), lambda i: (0, 0)),
            pl.BlockSpec((1, HID), lambda i: (0, 0)),
            pl.BlockSpec((1, 1, CH), lambda i: (i, 0, 0)),
            pl.BlockSpec((2 * HID, HID), lambda i: (0, 0)),
            pl.BlockSpec((1, HID), lambda i: (0, 0)),
            pl.BlockSpec((HID, 128), lambda i: (0, 0)),
            pl.BlockSpec((1, 128), lambda i: (0, 0)),
        ],
        out_specs=pl.BlockSpec((G, 128), lambda i: (0, 0)),
        out_shape=jax.ShapeDtypeStruct((G, 128), f32),
        scratch_shapes=[
            pltpu.VMEM((G, HID), f32),
            pltpu.VMEM((G, HID), f32),
            pltpu.VMEM((G, 128), f32),
        ],
    )(s_td2, s_td2, u_td2, u_td2, s_bu2, s_bu2, u_bu2, u_bu2,
      ditd, dibu, b_td2, b_bu2, batch3, W_h1, b_h1, W_h2p, b_h2p)


def _xla_agg(u3, src, dst, width):
    u_cat = u3.reshape(2 * N, width)
    src_cat = jnp.concatenate([src, src + N])
    dst_cat = jnp.concatenate([dst, dst + N])
    s = jnp.zeros((2 * N, width), u3.dtype).at[dst_cat].add(u_cat[src_cat])
    return s.reshape(2, N, width)


def kernel(x, edge_index, bot_up_edge_index, root_feat, text_feat, batch,
           W_td1, b_td1, W_td2, b_td2, W_bu1, b_bu1, W_bu2, b_bu2,
           W_h1, b_h1, W_h2, b_h2):
    f32 = jnp.float32
    src = edge_index[0]
    dst = edge_index[1]

    hist_td = jnp.zeros((N,), jnp.int32).at[dst].add(1)
    hist_bu = jnp.zeros((N,), jnp.int32).at[src].add(1)
    degp_td = jnp.zeros((DEGP, N), jnp.int32).at[0].set(hist_td).reshape(
        DEGP, NB, CH).transpose(1, 0, 2)
    degp_bu = jnp.zeros((DEGP, N), jnp.int32).at[0].set(hist_bu).reshape(
        DEGP, NB, CH).transpose(1, 0, 2)

    batch3 = batch.reshape(NB, 1, CH)
    u_td1, u_bu1, rf, ditd, dibu = _prep(
        x, text_feat, root_feat, batch3, degp_td, degp_bu)

    s_td1 = _xla_agg(u_td1, src, dst, 64)
    s_bu1 = _xla_agg(u_bu1, dst, src, 64)

    W_td2a, W_td2b = W_td2[:HID], W_td2[HID:]
    W_bu2a, W_bu2b = W_bu2[:HID], W_bu2[HID:]
    u_td2, u_bu2 = _mid(s_td1, u_td1, s_bu1, u_bu1, rf, ditd, dibu,
                        W_td1, b_td1.reshape(1, HID), W_td2a, W_td2b,
                        W_bu1, b_bu1.reshape(1, HID), W_bu2a, W_bu2b)

    s_td2 = _xla_agg(u_td2, src, dst, 128)
    s_bu2 = _xla_agg(u_bu2, dst, src, 128)

    W_h2p = jnp.pad(W_h2, ((0, 0), (0, 128 - NCLS)))
    b_h2p = jnp.pad(b_h2, (0, 128 - NCLS)).reshape(1, 128)
    outp = _final(s_td2, u_td2, s_bu2, u_bu2, ditd, dibu,
                  b_td2.reshape(1, HID), b_bu2.reshape(1, HID), batch3,
                  W_h1, b_h1.reshape(1, HID), W_h2p, b_h2p)
    return outp[:, :NCLS]


# speedup vs baseline: 1.6847x; 1.0177x over previous
import functools

import jax
import jax.numpy as jnp
from jax import lax
from jax.experimental import pallas as pl
from jax.experimental.pallas import tpu as pltpu
from jax.experimental.pallas import tpu_sc as plsc

N = 10000
E = 320000
G = 128
XD = 16
TD = 112
IND = 128
HID = 256
NCLS = 4
CH = 1000
NB = N // CH

NSUB = 16
CHW = 80
NROW = E // CHW
NROWP = 4096
NPADROW = NROWP - NROW
RPS = NROWP // NSUB
GR = 8
NGRP = RPS // GR
NPAD = 10240
HALF = 5120
NACC = HALF + 128


def _prep_body(x_ref, tf_ref, rt_ref, b3_ref, deg_ref,
               u1_ref, rf_ref, ditd_ref, dibu_ref):
    di_td = lax.rsqrt(deg_ref[0][:, 0:1] + 1.0)
    di_bu = lax.rsqrt(deg_ref[1][:, 0:1] + 1.0)
    batch = b3_ref[0, 0, :]
    gid = lax.broadcasted_iota(jnp.int32, (CH, G), 1)
    onehot = (batch[:, None] == gid).astype(jnp.float32)
    tf = jnp.dot(onehot, tf_ref[...], preferred_element_type=jnp.float32)
    rg = jnp.dot(onehot, rt_ref[...], preferred_element_type=jnp.float32)
    xf = jnp.concatenate([x_ref[...], tf], axis=1)
    rf_ref[...] = jnp.concatenate([rg, tf], axis=1)
    u1_ref[0] = xf * di_td
    u1_ref[1] = xf * di_bu
    ditd_ref[...] = jnp.broadcast_to(di_td, (CH, 8))
    dibu_ref[...] = jnp.broadcast_to(di_bu, (CH, 8))


def _prep(x, text_feat, root_feat, batch3, deg):
    f32 = jnp.float32
    return pl.pallas_call(
        _prep_body,
        grid=(NB,),
        in_specs=[
            pl.BlockSpec((CH, XD), lambda i: (i, 0)),
            pl.BlockSpec((G, TD), lambda i: (0, 0)),
            pl.BlockSpec((G, XD), lambda i: (0, 0)),
            pl.BlockSpec((1, 1, CH), lambda i: (i, 0, 0)),
            pl.BlockSpec((2, CH, 16), lambda i: (0, i, 0)),
        ],
        out_specs=[
            pl.BlockSpec((2, CH, IND), lambda i: (0, i, 0)),
            pl.BlockSpec((CH, IND), lambda i: (i, 0)),
            pl.BlockSpec((CH, 8), lambda i: (i, 0)),
            pl.BlockSpec((CH, 8), lambda i: (i, 0)),
        ],
        out_shape=[
            jax.ShapeDtypeStruct((2, N, IND), f32),
            jax.ShapeDtypeStruct((N, IND), f32),
            jax.ShapeDtypeStruct((N, 8), f32),
            jax.ShapeDtypeStruct((N, 8), f32),
        ],
    )(x, text_feat, root_feat, batch3, deg)


def _mid_body(std_ref, utd_ref, sbu_ref, ubu_ref,
              rf_ref, ditd_ref, dibu_ref,
              wtd1_ref, btd1_ref, wtd2a_ref, wtd2b_ref,
              wbu1_ref, bbu1_ref, wbu2a_ref, wbu2b_ref,
              otd_ref, obu_ref):
    rf = rf_ref[...]

    def one_dir(s_ref, u_ref, di_ref, w1, b1, w2a, w2b, out_ref):
        di = di_ref[:, 0:1]
        y = di * (s_ref[0] + u_ref[0])
        xh = jax.nn.relu(
            jnp.dot(y, w1[...], preferred_element_type=jnp.float32) + b1[0])
        u2 = di * (jnp.dot(xh, w2a[...], preferred_element_type=jnp.float32)
                   + jnp.dot(rf, w2b[...], preferred_element_type=jnp.float32))
        out_ref[0] = u2[:, :128]
        out_ref[1] = u2[:, 128:]

    one_dir(std_ref, utd_ref, ditd_ref, wtd1_ref, btd1_ref,
            wtd2a_ref, wtd2b_ref, otd_ref)
    one_dir(sbu_ref, ubu_ref, dibu_ref, wbu1_ref, bbu1_ref,
            wbu2a_ref, wbu2b_ref, obu_ref)


def _mid(s1, u1, rf, ditd, dibu,
         W_td1, b_td1, W_td2a, W_td2b, W_bu1, b_bu1, W_bu2a, W_bu2b):
    f32 = jnp.float32
    dirspec = lambda j: pl.BlockSpec((1, CH, IND), lambda i, j=j: (j, i, 0))
    full = lambda a, b: pl.BlockSpec((a, b), lambda i: (0, 0))
    return pl.pallas_call(
        _mid_body,
        grid=(NB,),
        in_specs=[
            dirspec(0), dirspec(0), dirspec(1), dirspec(1),
            pl.BlockSpec((CH, IND), lambda i: (i, 0)),
            pl.BlockSpec((CH, 8), lambda i: (i, 0)),
            pl.BlockSpec((CH, 8), lambda i: (i, 0)),
            full(IND, HID), pl.BlockSpec((1, HID), lambda i: (0, 0)),
            full(HID, HID), full(IND, HID),
            full(IND, HID), pl.BlockSpec((1, HID), lambda i: (0, 0)),
            full(HID, HID), full(IND, HID),
        ],
        out_specs=[
            pl.BlockSpec((2, CH, 128), lambda i: (0, i, 0)),
            pl.BlockSpec((2, CH, 128), lambda i: (0, i, 0)),
        ],
        out_shape=[
            jax.ShapeDtypeStruct((2, N, 128), f32),
            jax.ShapeDtypeStruct((2, N, 128), f32),
        ],
    )(s1, u1, s1, u1, rf, ditd, dibu, W_td1, b_td1, W_td2a, W_td2b,
      W_bu1, b_bu1, W_bu2a, W_bu2b)


def _final_body(stdl_ref, stdh_ref, utdl_ref, utdh_ref,
                sbul_ref, sbuh_ref, ubul_ref, ubuh_ref,
                ditd_ref, dibu_ref, btd2_ref, bbu2_ref, b3_ref,
                wh1_ref, bh1_ref, wh2_ref, bh2_ref,
                out_ref, acc_td, acc_bu, cnt):
    i = pl.program_id(0)

    @pl.when(i == 0)
    def _():
        acc_td[...] = jnp.zeros_like(acc_td)
        acc_bu[...] = jnp.zeros_like(acc_bu)
        cnt[...] = jnp.zeros_like(cnt)

    batch = b3_ref[0, 0, :]
    gid = lax.broadcasted_iota(jnp.int32, (G, CH), 0)
    mask = (batch[None, :] == gid).astype(jnp.float32)

    def one_dir(slo, shi, ulo, uhi, di_ref, b2):
        s = jnp.concatenate([slo[0], shi[0]], axis=1)
        u = jnp.concatenate([ulo[0], uhi[0]], axis=1)
        return jax.nn.relu(di_ref[:, 0:1] * (s + u) + b2[0])

    xtd2 = one_dir(stdl_ref, stdh_ref, utdl_ref, utdh_ref, ditd_ref, btd2_ref)
    xbu2 = one_dir(sbul_ref, sbuh_ref, ubul_ref, ubuh_ref, dibu_ref, bbu2_ref)
    acc_td[...] += jnp.dot(mask, xtd2, preferred_element_type=jnp.float32)
    acc_bu[...] += jnp.dot(mask, xbu2, preferred_element_type=jnp.float32)
    cnt[...] += jnp.dot(mask, jnp.ones((CH, 128), jnp.float32),
                        preferred_element_type=jnp.float32)

    @pl.when(i == NB - 1)
    def _():
        c = jnp.maximum(cnt[:, 0:1], 1.0)
        pooled = jnp.concatenate([acc_td[...] / c, acc_bu[...] / c], axis=1)
        h = jax.nn.relu(
            jnp.dot(pooled, wh1_ref[...], preferred_element_type=jnp.float32)
            + bh1_ref[0])
        out_ref[...] = (jnp.dot(h, wh2_ref[...],
                                preferred_element_type=jnp.float32)
                        + bh2_ref[0])


def _final(s_td2, u_td2, s_bu2, u_bu2, ditd, dibu, b_td2, b_bu2, batch3,
           W_h1, b_h1, W_h2p, b_h2p):
    f32 = jnp.float32
    half = lambda j: pl.BlockSpec((1, CH, 128), lambda i, j=j: (j, i, 0))
    return pl.pallas_call(
        _final_body,
        grid=(NB,),
        in_specs=[
            half(0), half(1), half(0), half(1),
            half(0), half(1), half(0), half(1),
            pl.BlockSpec((CH, 8), lambda i: (i, 0)),
            pl.BlockSpec((CH, 8), lambda i: (i, 0)),
            pl.BlockSpec((1, HID), lambda i: (0, 0)),
            pl.BlockSpec((1, HID), lambda i: (0, 0)),
            pl.BlockSpec((1, 1, CH), lambda i: (i, 0, 0)),
            pl.BlockSpec((2 * HID, HID), lambda i: (0, 0)),
            pl.BlockSpec((1, HID), lambda i: (0, 0)),
            pl.BlockSpec((HID, 128), lambda i: (0, 0)),
            pl.BlockSpec((1, 128), lambda i: (0, 0)),
        ],
        out_specs=pl.BlockSpec((G, 128), lambda i: (0, 0)),
        out_shape=jax.ShapeDtypeStruct((G, 128), f32),
        scratch_shapes=[
            pltpu.VMEM((G, HID), f32),
            pltpu.VMEM((G, HID), f32),
            pltpu.VMEM((G, 128), f32),
        ],
    )(s_td2, s_td2, u_td2, u_td2, s_bu2, s_bu2, u_bu2, u_bu2,
      ditd, dibu, b_td2, b_bu2, batch3, W_h1, b_h1, W_h2p, b_h2p)


def kernel(x, edge_index, bot_up_edge_index, root_feat, text_feat, batch,
           W_td1, b_td1, W_td2, b_td2, W_bu1, b_bu1, W_bu2, b_bu2,
           W_h1, b_h1, W_h2, b_h2):
    f32 = jnp.float32
    e2 = edge_index.reshape(2, NROW, CHW)
    pad_s = jnp.zeros((NPADROW, CHW), jnp.int32)
    pad_d = jnp.full((NPADROW, CHW), N, jnp.int32)
    esrc = jnp.concatenate([e2[0], pad_s], 0)
    edst = jnp.concatenate([e2[1], pad_s], 0)
    esrc_d = jnp.concatenate([e2[0], pad_d], 0)
    edst_d = jnp.concatenate([e2[1], pad_d], 0)

    hist_td = jnp.zeros((N,), f32).at[edge_index[1]].add(1.0)
    hist_bu = jnp.zeros((N,), f32).at[edge_index[0]].add(1.0)
    deg = jnp.stack([
        jnp.broadcast_to(hist_td[:, None], (N, 16)),
        jnp.broadcast_to(hist_bu[:, None], (N, 16))])

    batch3 = batch.reshape(NB, 1, CH)
    u1, rf, ditd, dibu = _prep(x, text_feat, root_feat, batch3, deg)

    def _remap(d2, h):
        lo = h * HALF
        return jnp.where((d2 >= lo) & (d2 < lo + HALF), d2 - lo, HALF)

    def _edge_agg(src2d, dst2d, u_cat, zeros, width):
        out = []
        for cc in range(2):
            sv = src2d[cc * NROWP:(cc + 1) * NROWP].reshape(-1)
            dv = dst2d[cc * NROWP:(cc + 1) * NROWP].reshape(-1)
            out.append(jnp.zeros((zeros.shape[0], width), f32).at[dv].add(
                u_cat[sv]))
        return jnp.stack(out)

    def _agg_halves(src2d, dst_a, dst_b, u_cat, zeros):
        parts = [_edge_agg(src2d,
                         jnp.concatenate([_remap(dst_a, h), _remap(dst_b, h)], 0),
                         u_cat, zeros, 128) for h in range(2)]
        return jnp.concatenate([parts[0][:, :HALF], parts[1][:, :N - HALF]],
                               axis=1)

    zerosA = jnp.zeros((NACC, 128), f32)

    src_l1 = jnp.concatenate([esrc, edst + N], 0)
    s1 = _agg_halves(src_l1, edst_d, esrc_d, u1.reshape(2 * N, IND), zerosA)

    W_td2a, W_td2b = W_td2[:HID], W_td2[HID:]
    W_bu2a, W_bu2b = W_bu2[:HID], W_bu2[HID:]
    u_td2, u_bu2 = _mid(s1, u1, rf, ditd, dibu,
                        W_td1, b_td1.reshape(1, HID), W_td2a, W_td2b,
                        W_bu1, b_bu1.reshape(1, HID), W_bu2a, W_bu2b)

    src_td = jnp.concatenate([esrc, esrc + N], 0)
    src_bu = jnp.concatenate([edst, edst + N], 0)
    s_td2 = _agg_halves(src_td, edst_d, edst_d, u_td2.reshape(2 * N, 128),
                        zerosA)
    s_bu2 = _agg_halves(src_bu, esrc_d, esrc_d, u_bu2.reshape(2 * N, 128),
                        zerosA)

    W_h2p = jnp.pad(W_h2, ((0, 0), (0, 128 - NCLS)))
    b_h2p = jnp.pad(b_h2, (0, 128 - NCLS)).reshape(1, 128)
    outp = _final(s_td2, u_td2, s_bu2, u_bu2, ditd, dibu,
                  b_td2.reshape(1, HID), b_bu2.reshape(1, HID), batch3,
                  W_h1, b_h1.reshape(1, HID), W_h2p, b_h2p)
    return outp[:, :NCLS]


# speedup vs baseline: 1.6870x; 1.0014x over previous
import jax
import jax.numpy as jnp
from jax import lax
from jax.experimental import pallas as pl
from jax.experimental.pallas import tpu as pltpu

N = 10000
E = 320000
G = 128
XD = 16
TD = 112
IND = 128
HID = 256
NCLS = 4
CH = 1000
NB = N // CH

NSUB = 16
CHW = 80
NROW = E // CHW
NROWP = 4096
NPADROW = NROWP - NROW
RPS = NROWP // NSUB
GR = 8
NGRP = RPS // GR
NPAD = 10240
HALF = 5120
NACC = HALF + 128


def _prep_body(x_ref, tf_ref, rt_ref, b3_ref, deg_ref,
               u1_ref, rf_ref, ditd_ref, dibu_ref):
    di_td = lax.rsqrt(deg_ref[0][:, 0:1] + 1.0)
    di_bu = lax.rsqrt(deg_ref[1][:, 0:1] + 1.0)
    batch = b3_ref[0, 0, :]
    gid = lax.broadcasted_iota(jnp.int32, (CH, G), 1)
    onehot = (batch[:, None] == gid).astype(jnp.float32)
    tf = jnp.dot(onehot, tf_ref[...], preferred_element_type=jnp.float32)
    rg = jnp.dot(onehot, rt_ref[...], preferred_element_type=jnp.float32)
    xf = jnp.concatenate([x_ref[...], tf], axis=1)
    rf_ref[...] = jnp.concatenate([rg, tf], axis=1)
    u1_ref[0] = xf * di_td
    u1_ref[1] = xf * di_bu
    ditd_ref[...] = jnp.broadcast_to(di_td, (CH, 8))
    dibu_ref[...] = jnp.broadcast_to(di_bu, (CH, 8))


def _prep(x, text_feat, root_feat, batch3, deg):
    f32 = jnp.float32
    return pl.pallas_call(
        _prep_body,
        grid=(NB,),
        in_specs=[
            pl.BlockSpec((CH, XD), lambda i: (i, 0)),
            pl.BlockSpec((G, TD), lambda i: (0, 0)),
            pl.BlockSpec((G, XD), lambda i: (0, 0)),
            pl.BlockSpec((1, 1, CH), lambda i: (i, 0, 0)),
            pl.BlockSpec((2, CH, 16), lambda i: (0, i, 0)),
        ],
        out_specs=[
            pl.BlockSpec((2, CH, IND), lambda i: (0, i, 0)),
            pl.BlockSpec((CH, IND), lambda i: (i, 0)),
            pl.BlockSpec((CH, 8), lambda i: (i, 0)),
            pl.BlockSpec((CH, 8), lambda i: (i, 0)),
        ],
        out_shape=[
            jax.ShapeDtypeStruct((2, N, IND), f32),
            jax.ShapeDtypeStruct((N, IND), f32),
            jax.ShapeDtypeStruct((N, 8), f32),
            jax.ShapeDtypeStruct((N, 8), f32),
        ],
    )(x, text_feat, root_feat, batch3, deg)


def _mid_body(std_ref, utd_ref, sbu_ref, ubu_ref,
              rf_ref, ditd_ref, dibu_ref,
              wtd1_ref, btd1_ref, wtd2a_ref, wtd2b_ref,
              wbu1_ref, bbu1_ref, wbu2a_ref, wbu2b_ref,
              otd_ref, obu_ref):
    rf = rf_ref[...]

    def one_dir(s_ref, u_ref, di_ref, w1, b1, w2a, w2b, out_ref):
        di = di_ref[:, 0:1]
        y = di * (s_ref[0] + u_ref[0])
        xh = jax.nn.relu(
            jnp.dot(y, w1[...], preferred_element_type=jnp.float32) + b1[0])
        u2 = di * (jnp.dot(xh, w2a[...], preferred_element_type=jnp.float32)
                   + jnp.dot(rf, w2b[...], preferred_element_type=jnp.float32))
        out_ref[0] = u2[:, :128]
        out_ref[1] = u2[:, 128:]

    one_dir(std_ref, utd_ref, ditd_ref, wtd1_ref, btd1_ref,
            wtd2a_ref, wtd2b_ref, otd_ref)
    one_dir(sbu_ref, ubu_ref, dibu_ref, wbu1_ref, bbu1_ref,
            wbu2a_ref, wbu2b_ref, obu_ref)


def _mid(s1, u1, rf, ditd, dibu,
         W_td1, b_td1, W_td2a, W_td2b, W_bu1, b_bu1, W_bu2a, W_bu2b):
    f32 = jnp.float32
    dirspec = lambda j: pl.BlockSpec((1, CH, IND), lambda i, j=j: (j, i, 0))
    full = lambda a, b: pl.BlockSpec((a, b), lambda i: (0, 0))
    return pl.pallas_call(
        _mid_body,
        grid=(NB,),
        in_specs=[
            dirspec(0), dirspec(0), dirspec(1), dirspec(1),
            pl.BlockSpec((CH, IND), lambda i: (i, 0)),
            pl.BlockSpec((CH, 8), lambda i: (i, 0)),
            pl.BlockSpec((CH, 8), lambda i: (i, 0)),
            full(IND, HID), pl.BlockSpec((1, HID), lambda i: (0, 0)),
            full(HID, HID), full(IND, HID),
            full(IND, HID), pl.BlockSpec((1, HID), lambda i: (0, 0)),
            full(HID, HID), full(IND, HID),
        ],
        out_specs=[
            pl.BlockSpec((2, CH, 128), lambda i: (0, i, 0)),
            pl.BlockSpec((2, CH, 128), lambda i: (0, i, 0)),
        ],
        out_shape=[
            jax.ShapeDtypeStruct((2, N, 128), f32),
            jax.ShapeDtypeStruct((2, N, 128), f32),
        ],
    )(s1, u1, s1, u1, rf, ditd, dibu, W_td1, b_td1, W_td2a, W_td2b,
      W_bu1, b_bu1, W_bu2a, W_bu2b)


def _final_body(stdl_ref, stdh_ref, utdl_ref, utdh_ref,
                sbul_ref, sbuh_ref, ubul_ref, ubuh_ref,
                ditd_ref, dibu_ref, btd2_ref, bbu2_ref, b3_ref,
                wh1_ref, bh1_ref, wh2_ref, bh2_ref,
                out_ref, acc_td, acc_bu, cnt):
    i = pl.program_id(0)

    @pl.when(i == 0)
    def _():
        acc_td[...] = jnp.zeros_like(acc_td)
        acc_bu[...] = jnp.zeros_like(acc_bu)
        cnt[...] = jnp.zeros_like(cnt)

    batch = b3_ref[0, 0, :]
    gid = lax.broadcasted_iota(jnp.int32, (G, CH), 0)
    mask = (batch[None, :] == gid).astype(jnp.float32)

    def one_dir(slo, shi, ulo, uhi, di_ref, b2):
        s = jnp.concatenate([slo[0], shi[0]], axis=1)
        u = jnp.concatenate([ulo[0], uhi[0]], axis=1)
        return jax.nn.relu(di_ref[:, 0:1] * (s + u) + b2[0])

    xtd2 = one_dir(stdl_ref, stdh_ref, utdl_ref, utdh_ref, ditd_ref, btd2_ref)
    xbu2 = one_dir(sbul_ref, sbuh_ref, ubul_ref, ubuh_ref, dibu_ref, bbu2_ref)
    acc_td[...] += jnp.dot(mask, xtd2, preferred_element_type=jnp.float32)
    acc_bu[...] += jnp.dot(mask, xbu2, preferred_element_type=jnp.float32)
    cnt[...] += jnp.dot(mask, jnp.ones((CH, 128), jnp.float32),
                        preferred_element_type=jnp.float32)

    @pl.when(i == NB - 1)
    def _():
        c = jnp.maximum(cnt[:, 0:1], 1.0)
        pooled = jnp.concatenate([acc_td[...] / c, acc_bu[...] / c], axis=1)
        h = jax.nn.relu(
            jnp.dot(pooled, wh1_ref[...], preferred_element_type=jnp.float32)
            + bh1_ref[0])
        out_ref[...] = (jnp.dot(h, wh2_ref[...],
                                preferred_element_type=jnp.float32)
                        + bh2_ref[0])


def _final(s_td2, u_td2, s_bu2, u_bu2, ditd, dibu, b_td2, b_bu2, batch3,
           W_h1, b_h1, W_h2p, b_h2p):
    f32 = jnp.float32
    half = lambda j: pl.BlockSpec((1, CH, 128), lambda i, j=j: (j, i, 0))
    return pl.pallas_call(
        _final_body,
        grid=(NB,),
        in_specs=[
            half(0), half(1), half(0), half(1),
            half(0), half(1), half(0), half(1),
            pl.BlockSpec((CH, 8), lambda i: (i, 0)),
            pl.BlockSpec((CH, 8), lambda i: (i, 0)),
            pl.BlockSpec((1, HID), lambda i: (0, 0)),
            pl.BlockSpec((1, HID), lambda i: (0, 0)),
            pl.BlockSpec((1, 1, CH), lambda i: (i, 0, 0)),
            pl.BlockSpec((2 * HID, HID), lambda i: (0, 0)),
            pl.BlockSpec((1, HID), lambda i: (0, 0)),
            pl.BlockSpec((HID, 128), lambda i: (0, 0)),
            pl.BlockSpec((1, 128), lambda i: (0, 0)),
        ],
        out_specs=pl.BlockSpec((G, 128), lambda i: (0, 0)),
        out_shape=jax.ShapeDtypeStruct((G, 128), f32),
        scratch_shapes=[
            pltpu.VMEM((G, HID), f32),
            pltpu.VMEM((G, HID), f32),
            pltpu.VMEM((G, 128), f32),
        ],
    )(s_td2, s_td2, u_td2, u_td2, s_bu2, s_bu2, u_bu2, u_bu2,
      ditd, dibu, b_td2, b_bu2, batch3, W_h1, b_h1, W_h2p, b_h2p)


def kernel(x, edge_index, bot_up_edge_index, root_feat, text_feat, batch,
           W_td1, b_td1, W_td2, b_td2, W_bu1, b_bu1, W_bu2, b_bu2,
           W_h1, b_h1, W_h2, b_h2):
    f32 = jnp.float32
    e2 = edge_index.reshape(2, NROW, CHW)
    pad_s = jnp.zeros((NPADROW, CHW), jnp.int32)
    pad_d = jnp.full((NPADROW, CHW), N, jnp.int32)
    esrc = jnp.concatenate([e2[0], pad_s], 0)
    edst = jnp.concatenate([e2[1], pad_s], 0)
    esrc_d = jnp.concatenate([e2[0], pad_d], 0)
    edst_d = jnp.concatenate([e2[1], pad_d], 0)

    hist_td = jnp.zeros((N,), f32).at[edge_index[1]].add(1.0)
    hist_bu = jnp.zeros((N,), f32).at[edge_index[0]].add(1.0)
    deg = jnp.stack([
        jnp.broadcast_to(hist_td[:, None], (N, 16)),
        jnp.broadcast_to(hist_bu[:, None], (N, 16))])

    batch3 = batch.reshape(NB, 1, CH)
    u1, rf, ditd, dibu = _prep(x, text_feat, root_feat, batch3, deg)

    def _remap(d2, h):
        lo = h * HALF
        return jnp.where((d2 >= lo) & (d2 < lo + HALF), d2 - lo, HALF)

    def _edge_agg(src2d, dst2d, u_cat, zeros, width):
        out = []
        for cc in range(2):
            sv = src2d[cc * NROWP:(cc + 1) * NROWP].reshape(-1)
            dv = dst2d[cc * NROWP:(cc + 1) * NROWP].reshape(-1)
            out.append(jnp.zeros((zeros.shape[0], width), f32).at[dv].add(
                u_cat[sv]))
        return jnp.stack(out)

    def _agg_halves(src2d, dst_a, dst_b, u_cat, zeros):
        parts = [_edge_agg(src2d,
                         jnp.concatenate([_remap(dst_a, h), _remap(dst_b, h)], 0),
                         u_cat, zeros, 128) for h in range(2)]
        return jnp.concatenate([parts[0][:, :HALF], parts[1][:, :N - HALF]],
                               axis=1)

    zerosA = jnp.zeros((NACC, 128), f32)

    src_l1 = jnp.concatenate([esrc, edst + N], 0)
    s1 = _agg_halves(src_l1, edst_d, esrc_d, u1.reshape(2 * N, IND), zerosA)

    W_td2a, W_td2b = W_td2[:HID], W_td2[HID:]
    W_bu2a, W_bu2b = W_bu2[:HID], W_bu2[HID:]
    u_td2, u_bu2 = _mid(s1, u1, rf, ditd, dibu,
                        W_td1, b_td1.reshape(1, HID), W_td2a, W_td2b,
                        W_bu1, b_bu1.reshape(1, HID), W_bu2a, W_bu2b)

    src_td = jnp.concatenate([esrc, esrc + N], 0)
    src_bu = jnp.concatenate([edst, edst + N], 0)
    s_td2 = _agg_halves(src_td, edst_d, edst_d, u_td2.reshape(2 * N, 128),
                        zerosA)
    s_bu2 = _agg_halves(src_bu, esrc_d, esrc_d, u_bu2.reshape(2 * N, 128),
                        zerosA)

    W_h2p = jnp.pad(W_h2, ((0, 0), (0, 128 - NCLS)))
    b_h2p = jnp.pad(b_h2, (0, 128 - NCLS)).reshape(1, 128)
    outp = _final(s_td2, u_td2, s_bu2, u_bu2, ditd, dibu,
                  b_td2.reshape(1, HID), b_bu2.reshape(1, HID), batch3,
                  W_h1, b_h1.reshape(1, HID), W_h2p, b_h2p)
    return outp[:, :NCLS]
